# baseline (speedup 1.0000x reference)
import functools

import jax
import jax.numpy as jnp
from jax import lax
from jax.experimental import pallas as pl
from jax.experimental.pallas import tpu as pltpu
from jax.experimental.pallas import tpu_sc as plsc

D = 128
B = 4096
L = 20

NC, NS = 2, 16
NW = NC * NS
RPW = B // NW
CHUNK = 4
NCHUNK = RPW // CHUNK
IDXC = CHUNK * L


NBUF = 4


def _pool_body(idx_hbm, table_hbm, out_hbm, idx_v, rows_bufs, out_v, sems):
    wid = lax.axis_index("s") * NC + lax.axis_index("c")
    pltpu.sync_copy(idx_hbm.at[wid], idx_v)
    for b in range(NBUF):
        pltpu.async_copy(table_hbm.at[idx_v.at[b]], rows_bufs[b], sems[b])

    def step(j, rows_v, sem):
        pltpu.make_async_copy(table_hbm.at[idx_v.at[j]], rows_v, sem).wait()
        inv = 1.0 / L
        for r in range(CHUNK):
            for d in range(D // 16):
                acc = rows_v[r * L, pl.ds(d * 16, 16)]
                for s in range(1, L):
                    acc = acc + rows_v[r * L + s, pl.ds(d * 16, 16)]
                out_v[j * CHUNK + r, pl.ds(d * 16, 16)] = acc * inv

        @pl.when(j + NBUF < NCHUNK)
        def _():
            pltpu.async_copy(table_hbm.at[idx_v.at[j + NBUF]], rows_v, sem)

    def ring_body(i, carry):
        for b in range(NBUF):
            step(NBUF * i + b, rows_bufs[b], sems[b])
        return carry

    lax.fori_loop(0, NCHUNK // NBUF, ring_body, 0)
    pltpu.sync_copy(out_v, out_hbm.at[pl.ds(wid * RPW, RPW)])


_pool = pl.kernel(
    _pool_body,
    out_type=jax.ShapeDtypeStruct((B, D), jnp.float32),
    mesh=plsc.VectorSubcoreMesh(core_axis_name="c", subcore_axis_name="s"),
    scratch_types=[
        pltpu.VMEM((NCHUNK, IDXC), jnp.int32),
        [pltpu.VMEM((IDXC, D), jnp.float32) for _ in range(NBUF)],
        pltpu.VMEM((RPW, D), jnp.float32),
        [pltpu.SemaphoreType.DMA for _ in range(NBUF)],
    ],
)


BLK = 512


def _proj_body(x_ref, w_ref, b_ref, o_ref):
    y = jnp.dot(x_ref[...], w_ref[...], preferred_element_type=jnp.float32)
    y = y + b_ref[...]
    s = jnp.sum(y * y, axis=1, keepdims=True)
    o_ref[...] = y * lax.rsqrt(s)


_proj = pl.pallas_call(
    _proj_body,
    grid=(B // BLK,),
    in_specs=[
        pl.BlockSpec((BLK, D), lambda i: (i, 0)),
        pl.BlockSpec((D, D), lambda i: (0, 0)),
        pl.BlockSpec((1, D), lambda i: (0, 0)),
    ],
    out_specs=pl.BlockSpec((BLK, D), lambda i: (i, 0)),
    out_shape=jax.ShapeDtypeStruct((B, D), jnp.float32),
)


def kernel(inputs, table, W, b):
    idx = inputs.astype(jnp.int32).reshape(NW, NCHUNK, IDXC)
    pooled = _pool(idx, table)
    return _proj(pooled, W, b.reshape(1, D))


# speedup vs baseline: 1.4781x; 1.4781x over previous
import functools

import jax
import jax.numpy as jnp
from jax import lax
from jax.experimental import pallas as pl
from jax.experimental.pallas import tpu as pltpu
from jax.experimental.pallas import tpu_sc as plsc

D = 128
B = 4096
L = 20

NC, NS = 2, 16
NW = NC * NS
RPW = B // NW
CHUNK = 4
NCHUNK = RPW // CHUNK
IDXC = CHUNK * L


def _pool_body(idx_hbm, table_hbm, out_hbm, idx_v, rows_v, out_v, sem):
    wid = lax.axis_index("s") * NC + lax.axis_index("c")
    pltpu.sync_copy(idx_hbm.at[wid], idx_v)
    nd = D // 16

    def chunk_body(j, carry):
        pltpu.async_copy(table_hbm.at[idx_v.at[j]], rows_v, sem).wait()
        inv = 1.0 / L
        for r0 in range(0, CHUNK, 1):
            rows = (r0,)
            acc = [[rows_v[r * L, pl.ds(d * 16, 16)] for d in range(nd)]
                   for r in rows]
            for s in range(1, L):
                for k, r in enumerate(rows):
                    for d in range(nd):
                        acc[k][d] = acc[k][d] + rows_v[r * L + s,
                                                       pl.ds(d * 16, 16)]
            for k, r in enumerate(rows):
                for d in range(nd):
                    out_v[j * CHUNK + r, pl.ds(d * 16, 16)] = acc[k][d] * inv
        return carry

    lax.fori_loop(0, NCHUNK, chunk_body, 0)
    pltpu.sync_copy(out_v, out_hbm.at[pl.ds(wid * RPW, RPW)])


_pool = pl.kernel(
    _pool_body,
    out_type=jax.ShapeDtypeStruct((B, D), jnp.float32),
    mesh=plsc.VectorSubcoreMesh(core_axis_name="c", subcore_axis_name="s"),
    scratch_types=[
        pltpu.VMEM((NCHUNK, IDXC), jnp.int32),
        pltpu.VMEM((IDXC, D), jnp.float32),
        pltpu.VMEM((RPW, D), jnp.float32),
        pltpu.SemaphoreType.DMA,
    ],
)


BLK = 512


def _proj_body(x_ref, w_ref, b_ref, o_ref):
    y = jnp.dot(x_ref[...], w_ref[...], preferred_element_type=jnp.float32)
    y = y + b_ref[...]
    s = jnp.sum(y * y, axis=1, keepdims=True)
    o_ref[...] = y * lax.rsqrt(s)


_proj = pl.pallas_call(
    _proj_body,
    grid=(B // BLK,),
    in_specs=[
        pl.BlockSpec((BLK, D), lambda i: (i, 0)),
        pl.BlockSpec((D, D), lambda i: (0, 0)),
        pl.BlockSpec((1, D), lambda i: (0, 0)),
    ],
    out_specs=pl.BlockSpec((BLK, D), lambda i: (i, 0)),
    out_shape=jax.ShapeDtypeStruct((B, D), jnp.float32),
)


def kernel(inputs, table, W, b):
    idx = inputs.astype(jnp.int32).reshape(NW, NCHUNK, IDXC)
    pooled = _pool(idx, table)
    return _proj(pooled, W, b.reshape(1, D))


# speedup vs baseline: 1.6562x; 1.1205x over previous
import functools

import jax
import jax.numpy as jnp
from jax import lax
from jax.experimental import pallas as pl
from jax.experimental.pallas import tpu as pltpu
from jax.experimental.pallas import tpu_sc as plsc

D = 128
B = 4096
L = 20

NC, NS = 2, 16
NW = NC * NS
RPW = B // NW
CHUNK = 4
NCHUNK = RPW // CHUNK
IDXC = CHUNK * L


def _pool_body(idx_hbm, table_hbm, out_hbm, idx_v, rows0, rows1, out_v, sem0,
               sem1):
    wid = lax.axis_index("s") * NC + lax.axis_index("c")
    pltpu.sync_copy(idx_hbm.at[wid], idx_v)
    nd = D // 16
    pltpu.async_copy(table_hbm.at[idx_v.at[0]], rows0, sem0)
    pltpu.async_copy(table_hbm.at[idx_v.at[1]], rows1, sem1)

    def step(j, rows_v, sem):
        pltpu.make_async_copy(table_hbm.at[idx_v.at[j]], rows_v, sem).wait()
        inv = 1.0 / L
        for r0 in range(0, CHUNK, 1):
            rows = (r0,)
            acc = [[rows_v[r * L, pl.ds(d * 16, 16)] for d in range(nd)]
                   for r in rows]
            for s in range(1, L):
                for k, r in enumerate(rows):
                    for d in range(nd):
                        acc[k][d] = acc[k][d] + rows_v[r * L + s,
                                                       pl.ds(d * 16, 16)]
            for k, r in enumerate(rows):
                for d in range(nd):
                    out_v[j * CHUNK + r, pl.ds(d * 16, 16)] = acc[k][d] * inv

        @pl.when(j + 2 < NCHUNK)
        def _():
            pltpu.async_copy(table_hbm.at[idx_v.at[j + 2]], rows_v, sem)

    def pair_body(i, carry):
        step(2 * i, rows0, sem0)
        step(2 * i + 1, rows1, sem1)
        return carry

    lax.fori_loop(0, NCHUNK // 2, pair_body, 0)
    pltpu.sync_copy(out_v, out_hbm.at[pl.ds(wid * RPW, RPW)])


_pool = pl.kernel(
    _pool_body,
    out_type=jax.ShapeDtypeStruct((B, D), jnp.float32),
    mesh=plsc.VectorSubcoreMesh(core_axis_name="c", subcore_axis_name="s"),
    scratch_types=[
        pltpu.VMEM((NCHUNK, IDXC), jnp.int32),
        pltpu.VMEM((IDXC, D), jnp.float32),
        pltpu.VMEM((IDXC, D), jnp.float32),
        pltpu.VMEM((RPW, D), jnp.float32),
        pltpu.SemaphoreType.DMA,
        pltpu.SemaphoreType.DMA,
    ],
)


BLK = 512


def _proj_body(x_ref, w_ref, b_ref, o_ref):
    y = jnp.dot(x_ref[...], w_ref[...], preferred_element_type=jnp.float32)
    y = y + b_ref[...]
    s = jnp.sum(y * y, axis=1, keepdims=True)
    o_ref[...] = y * lax.rsqrt(s)


_proj = pl.pallas_call(
    _proj_body,
    grid=(B // BLK,),
    in_specs=[
        pl.BlockSpec((BLK, D), lambda i: (i, 0)),
        pl.BlockSpec((D, D), lambda i: (0, 0)),
        pl.BlockSpec((1, D), lambda i: (0, 0)),
    ],
    out_specs=pl.BlockSpec((BLK, D), lambda i: (i, 0)),
    out_shape=jax.ShapeDtypeStruct((B, D), jnp.float32),
)


def kernel(inputs, table, W, b):
    idx = inputs.astype(jnp.int32).reshape(NW, NCHUNK, IDXC)
    pooled = _pool(idx, table)
    return _proj(pooled, W, b.reshape(1, D))


# speedup vs baseline: 1.9226x; 1.1608x over previous
import functools

import jax
import jax.numpy as jnp
import numpy as np
from jax import lax
from jax.experimental import pallas as pl
from jax.experimental.pallas import tpu as pltpu
from jax.experimental.pallas import tpu_sc as plsc

D = 128
B = 4096
L = 20

NC, NS = 2, 16
NW = NC * NS
RPW = B // NW
CHUNK = 4
NCHUNK = RPW // CHUNK
IDXC = CHUNK * L


def _pool_body(idx_hbm, dpat_hbm, table_hbm, out_hbm, idx_v, dest_v, rows0,
               rows1, out_v, acc_sh, g0, g1, s0, s1):
    sid = lax.axis_index("s")
    wid = sid * NC + lax.axis_index("c")
    pltpu.sync_copy(idx_hbm.at[wid], idx_v)
    pltpu.sync_copy(dpat_hbm, dest_v)
    base = sid * RPW
    zeros = jnp.zeros((16,), jnp.float32)
    for r in range(RPW):
        for d in range(D // 16):
            out_v[r, pl.ds(d * 16, 16)] = zeros
    for j in range(NCHUNK):
        for i in range(IDXC // 16):
            dest_v[j, pl.ds(i * 16, 16)] = dest_v[j, pl.ds(i * 16, 16)] + base
    pltpu.sync_copy(out_v, acc_sh.at[pl.ds(sid * RPW, RPW)])

    pltpu.async_copy(table_hbm.at[idx_v.at[0]], rows0, g0)
    pltpu.async_copy(table_hbm.at[idx_v.at[1]], rows1, g1)

    def step(j, rows_v, gsem, ssem):
        pltpu.make_async_copy(table_hbm.at[idx_v.at[j]], rows_v, gsem).wait()
        pltpu.async_copy(rows_v, acc_sh.at[dest_v.at[j]], ssem, add=True)
        @pl.when(j + 2 < NCHUNK)
        def _():
            pltpu.make_async_copy(rows_v, acc_sh.at[dest_v.at[j]],
                                  ssem).wait()
            pltpu.async_copy(table_hbm.at[idx_v.at[j + 2]], rows_v, gsem)

    def pair_body(i, carry):
        step(2 * i, rows0, g0, s0)
        step(2 * i + 1, rows1, g1, s1)
        return carry

    lax.fori_loop(0, NCHUNK // 2, pair_body, 0)
    pltpu.make_async_copy(rows0, acc_sh.at[dest_v.at[NCHUNK - 2]], s0).wait()
    pltpu.make_async_copy(rows1, acc_sh.at[dest_v.at[NCHUNK - 1]], s1).wait()
    pltpu.sync_copy(acc_sh.at[pl.ds(sid * RPW, RPW)],
                    out_hbm.at[pl.ds(wid * RPW, RPW)])


_pool = pl.kernel(
    _pool_body,
    out_type=jax.ShapeDtypeStruct((B, D), jnp.float32),
    mesh=plsc.VectorSubcoreMesh(core_axis_name="c", subcore_axis_name="s"),
    scratch_types=[
        pltpu.VMEM((NCHUNK, IDXC), jnp.int32),
        pltpu.VMEM((NCHUNK, IDXC), jnp.int32),
        pltpu.VMEM((IDXC, D), jnp.float32),
        pltpu.VMEM((IDXC, D), jnp.float32),
        pltpu.VMEM((RPW, D), jnp.float32),
        pltpu.VMEM_SHARED((NS * RPW, D), jnp.float32),
        pltpu.SemaphoreType.DMA,
        pltpu.SemaphoreType.DMA,
        pltpu.SemaphoreType.DMA,
        pltpu.SemaphoreType.DMA,
    ],
)


BLK = 512


def _proj_body(x_ref, w_ref, b_ref, o_ref):
    y = jnp.dot(x_ref[...], w_ref[...], preferred_element_type=jnp.float32)
    y = y * (1.0 / L) + b_ref[...]
    s = jnp.sum(y * y, axis=1, keepdims=True)
    o_ref[...] = y * lax.rsqrt(s)


_proj = pl.pallas_call(
    _proj_body,
    grid=(B // BLK,),
    in_specs=[
        pl.BlockSpec((BLK, D), lambda i: (i, 0)),
        pl.BlockSpec((D, D), lambda i: (0, 0)),
        pl.BlockSpec((1, D), lambda i: (0, 0)),
    ],
    out_specs=pl.BlockSpec((BLK, D), lambda i: (i, 0)),
    out_shape=jax.ShapeDtypeStruct((B, D), jnp.float32),
)


_DEST_PATTERN = (np.arange(NCHUNK * IDXC, dtype=np.int32) // L).reshape(
    NCHUNK, IDXC)


def kernel(inputs, table, W, b):
    idx = inputs.astype(jnp.int32).reshape(NW, NCHUNK, IDXC)
    pooled = _pool(idx, jnp.asarray(_DEST_PATTERN), table)
    return _proj(pooled, W, b.reshape(1, D))


# speedup vs baseline: 1.9456x; 1.0120x over previous
import functools

import jax
import jax.numpy as jnp
import numpy as np
from jax import lax
from jax.experimental import pallas as pl
from jax.experimental.pallas import tpu as pltpu
from jax.experimental.pallas import tpu_sc as plsc

D = 128
B = 4096
L = 20

NC, NS = 2, 16
NW = NC * NS
RPW = B // NW
IDXC = 128
NCHUNK = (RPW * L) // IDXC


def _pool_body(idx_hbm, dpat_hbm, table_hbm, out_hbm, idx_v, dest_v, rows0,
               rows1, out_v, acc_sh, g0, g1, s0, s1):
    sid = lax.axis_index("s")
    wid = sid * NC + lax.axis_index("c")
    pltpu.sync_copy(idx_hbm.at[wid], idx_v)
    pltpu.sync_copy(dpat_hbm, dest_v)
    base = sid * RPW
    zeros = jnp.zeros((16,), jnp.float32)
    for r in range(RPW):
        for d in range(D // 16):
            out_v[r, pl.ds(d * 16, 16)] = zeros
    for j in range(NCHUNK):
        for i in range(IDXC // 16):
            dest_v[j, pl.ds(i * 16, 16)] = dest_v[j, pl.ds(i * 16, 16)] + base
    pltpu.sync_copy(out_v, acc_sh.at[pl.ds(sid * RPW, RPW)])

    pltpu.async_copy(table_hbm.at[idx_v.at[0]], rows0, g0)
    pltpu.async_copy(table_hbm.at[idx_v.at[1]], rows1, g1)

    def step(j, rows_v, gsem, ssem):
        pltpu.make_async_copy(table_hbm.at[idx_v.at[j]], rows_v, gsem).wait()
        pltpu.async_copy(rows_v, acc_sh.at[dest_v.at[j]], ssem, add=True)
        @pl.when(j + 2 < NCHUNK)
        def _():
            pltpu.make_async_copy(rows_v, acc_sh.at[dest_v.at[j]],
                                  ssem).wait()
            pltpu.async_copy(table_hbm.at[idx_v.at[j + 2]], rows_v, gsem)

    def pair_body(i, carry):
        step(2 * i, rows0, g0, s0)
        step(2 * i + 1, rows1, g1, s1)
        return carry

    lax.fori_loop(0, NCHUNK // 2, pair_body, 0)
    pltpu.make_async_copy(rows0, acc_sh.at[dest_v.at[NCHUNK - 2]], s0).wait()
    pltpu.make_async_copy(rows1, acc_sh.at[dest_v.at[NCHUNK - 1]], s1).wait()
    pltpu.sync_copy(acc_sh.at[pl.ds(sid * RPW, RPW)],
                    out_hbm.at[pl.ds(wid * RPW, RPW)])


_pool = pl.kernel(
    _pool_body,
    out_type=jax.ShapeDtypeStruct((B, D), jnp.float32),
    mesh=plsc.VectorSubcoreMesh(core_axis_name="c", subcore_axis_name="s"),
    scratch_types=[
        pltpu.VMEM((NCHUNK, IDXC), jnp.int32),
        pltpu.VMEM((NCHUNK, IDXC), jnp.int32),
        pltpu.VMEM((IDXC, D), jnp.float32),
        pltpu.VMEM((IDXC, D), jnp.float32),
        pltpu.VMEM((RPW, D), jnp.float32),
        pltpu.VMEM_SHARED((NS * RPW, D), jnp.float32),
        pltpu.SemaphoreType.DMA,
        pltpu.SemaphoreType.DMA,
        pltpu.SemaphoreType.DMA,
        pltpu.SemaphoreType.DMA,
    ],
)


BLK = 512


def _proj_body(x_ref, w_ref, b_ref, o_ref):
    y = jnp.dot(x_ref[...], w_ref[...], preferred_element_type=jnp.float32)
    y = y * (1.0 / L) + b_ref[...]
    s = jnp.sum(y * y, axis=1, keepdims=True)
    o_ref[...] = y * lax.rsqrt(s)


_proj = pl.pallas_call(
    _proj_body,
    grid=(B // BLK,),
    in_specs=[
        pl.BlockSpec((BLK, D), lambda i: (i, 0)),
        pl.BlockSpec((D, D), lambda i: (0, 0)),
        pl.BlockSpec((1, D), lambda i: (0, 0)),
    ],
    out_specs=pl.BlockSpec((BLK, D), lambda i: (i, 0)),
    out_shape=jax.ShapeDtypeStruct((B, D), jnp.float32),
)


_DEST_PATTERN = (np.arange(NCHUNK * IDXC, dtype=np.int32) // L).reshape(
    NCHUNK, IDXC)


def kernel(inputs, table, W, b):
    idx = inputs.astype(jnp.int32).reshape(NW, NCHUNK, IDXC)
    pooled = _pool(idx, jnp.asarray(_DEST_PATTERN), table)
    return _proj(pooled, W, b.reshape(1, D))


# speedup vs baseline: 1.9749x; 1.0151x over previous
import functools

import jax
import jax.numpy as jnp
import numpy as np
from jax import lax
from jax.experimental import pallas as pl
from jax.experimental.pallas import tpu as pltpu
from jax.experimental.pallas import tpu_sc as plsc

D = 128
B = 4096
L = 20

NC, NS = 2, 16
NW = NC * NS
RPW = B // NW
IDXC = 128
NCHUNK = (RPW * L) // IDXC


NBUF = 4


def _pool_body(idx_hbm, dpat_hbm, table_hbm, out_hbm, idx_v, dest_v, bufs,
               out_v, acc_sh, gsems, ssems):
    sid = lax.axis_index("s")
    wid = sid * NC + lax.axis_index("c")
    pltpu.sync_copy(idx_hbm.at[wid], idx_v)
    pltpu.sync_copy(dpat_hbm, dest_v)
    base = sid * RPW
    zeros = jnp.zeros((16,), jnp.float32)
    for r in range(RPW):
        for d in range(D // 16):
            out_v[r, pl.ds(d * 16, 16)] = zeros
    for j in range(NCHUNK):
        for i in range(IDXC // 16):
            dest_v[j, pl.ds(i * 16, 16)] = dest_v[j, pl.ds(i * 16, 16)] + base
    pltpu.sync_copy(out_v, acc_sh.at[pl.ds(sid * RPW, RPW)])

    pltpu.async_copy(table_hbm.at[idx_v.at[0]], bufs[0], gsems[0])
    pltpu.async_copy(table_hbm.at[idx_v.at[1]], bufs[1], gsems[1])

    def step(j, k):
        b, b2 = k % NBUF, (k + 2) % NBUF
        pltpu.make_async_copy(table_hbm.at[idx_v.at[j]], bufs[b],
                              gsems[b]).wait()
        pltpu.async_copy(bufs[b], acc_sh.at[dest_v.at[j]], ssems[b], add=True)

        @pl.when(j + 2 < NCHUNK)
        def _():
            @pl.when(j >= 2)
            def _():
                pltpu.make_async_copy(bufs[b2], acc_sh.at[dest_v.at[j - 2]],
                                      ssems[b2]).wait()
            pltpu.async_copy(table_hbm.at[idx_v.at[j + 2]], bufs[b2],
                             gsems[b2])

    def quad_body(i, carry):
        for k in range(NBUF):
            step(NBUF * i + k, k)
        return carry

    lax.fori_loop(0, NCHUNK // NBUF, quad_body, 0)
    for jj in range(NCHUNK - NBUF, NCHUNK):
        pltpu.make_async_copy(bufs[jj % NBUF], acc_sh.at[dest_v.at[jj]],
                              ssems[jj % NBUF]).wait()
    pltpu.sync_copy(acc_sh.at[pl.ds(sid * RPW, RPW)],
                    out_hbm.at[pl.ds(wid * RPW, RPW)])


_pool = pl.kernel(
    _pool_body,
    out_type=jax.ShapeDtypeStruct((B, D), jnp.float32),
    mesh=plsc.VectorSubcoreMesh(core_axis_name="c", subcore_axis_name="s"),
    scratch_types=[
        pltpu.VMEM((NCHUNK, IDXC), jnp.int32),
        pltpu.VMEM((NCHUNK, IDXC), jnp.int32),
        [pltpu.VMEM((IDXC, D), jnp.float32) for _ in range(NBUF)],
        pltpu.VMEM((RPW, D), jnp.float32),
        pltpu.VMEM_SHARED((NS * RPW, D), jnp.float32),
        [pltpu.SemaphoreType.DMA for _ in range(NBUF)],
        [pltpu.SemaphoreType.DMA for _ in range(NBUF)],
    ],
)


BLK = 512


def _proj_body(x_ref, w_ref, b_ref, o_ref):
    y = jnp.dot(x_ref[...], w_ref[...], preferred_element_type=jnp.float32)
    y = y * (1.0 / L) + b_ref[...]
    s = jnp.sum(y * y, axis=1, keepdims=True)
    o_ref[...] = y * lax.rsqrt(s)


_proj = pl.pallas_call(
    _proj_body,
    grid=(B // BLK,),
    in_specs=[
        pl.BlockSpec((BLK, D), lambda i: (i, 0)),
        pl.BlockSpec((D, D), lambda i: (0, 0)),
        pl.BlockSpec((1, D), lambda i: (0, 0)),
    ],
    out_specs=pl.BlockSpec((BLK, D), lambda i: (i, 0)),
    out_shape=jax.ShapeDtypeStruct((B, D), jnp.float32),
)


_DEST_PATTERN = (np.arange(NCHUNK * IDXC, dtype=np.int32) // L).reshape(
    NCHUNK, IDXC)


def kernel(inputs, table, W, b):
    idx = inputs.astype(jnp.int32).reshape(NW, NCHUNK, IDXC)
    pooled = _pool(idx, jnp.asarray(_DEST_PATTERN), table)
    return _proj(pooled, W, b.reshape(1, D))


# speedup vs baseline: 2.0590x; 1.0426x over previous
import functools

import jax
import jax.numpy as jnp
import numpy as np
from jax import lax
from jax.experimental import pallas as pl
from jax.experimental.pallas import tpu as pltpu
from jax.experimental.pallas import tpu_sc as plsc

D = 128
B = 4096
L = 20

NC, NS = 2, 16
NW = NC * NS
RPW = B // NW
IDXC = 128
NCHUNK = (RPW * L) // IDXC


NBUF = 4


def _pool_body(idx_hbm, dpat_hbm, table_hbm, out_hbm, idx_v, dest_v, bufs,
               out_v, acc_sh, gsems, ssems):
    sid = lax.axis_index("s")
    wid = sid * NC + lax.axis_index("c")
    pltpu.sync_copy(idx_hbm.at[wid], idx_v)
    pltpu.async_copy(table_hbm.at[idx_v.at[0]], bufs[0], gsems[0])
    pltpu.async_copy(table_hbm.at[idx_v.at[1]], bufs[1], gsems[1])
    pltpu.sync_copy(dpat_hbm.at[sid], dest_v)
    zeros = jnp.zeros((16,), jnp.float32)
    for r in range(RPW):
        for d in range(D // 16):
            out_v[r, pl.ds(d * 16, 16)] = zeros
    pltpu.sync_copy(out_v, acc_sh.at[pl.ds(sid * RPW, RPW)])

    def step(j, k):
        b, b2 = k % NBUF, (k + 2) % NBUF
        pltpu.make_async_copy(table_hbm.at[idx_v.at[j]], bufs[b],
                              gsems[b]).wait()
        pltpu.async_copy(bufs[b], acc_sh.at[dest_v.at[j]], ssems[b], add=True)

        @pl.when(j + 2 < NCHUNK)
        def _():
            @pl.when(j >= 2)
            def _():
                pltpu.make_async_copy(bufs[b2], acc_sh.at[dest_v.at[j - 2]],
                                      ssems[b2]).wait()
            pltpu.async_copy(table_hbm.at[idx_v.at[j + 2]], bufs[b2],
                             gsems[b2])

    def quad_body(i, carry):
        for k in range(NBUF):
            step(NBUF * i + k, k)
        return carry

    lax.fori_loop(0, NCHUNK // NBUF, quad_body, 0)
    for jj in range(NCHUNK - NBUF, NCHUNK):
        pltpu.make_async_copy(bufs[jj % NBUF], acc_sh.at[dest_v.at[jj]],
                              ssems[jj % NBUF]).wait()
    pltpu.sync_copy(acc_sh.at[pl.ds(sid * RPW, RPW)],
                    out_hbm.at[pl.ds(wid * RPW, RPW)])


_pool = pl.kernel(
    _pool_body,
    out_type=jax.ShapeDtypeStruct((B, D), jnp.float32),
    mesh=plsc.VectorSubcoreMesh(core_axis_name="c", subcore_axis_name="s"),
    scratch_types=[
        pltpu.VMEM((NCHUNK, IDXC), jnp.int32),
        pltpu.VMEM((NCHUNK, IDXC), jnp.int32),
        [pltpu.VMEM((IDXC, D), jnp.float32) for _ in range(NBUF)],
        pltpu.VMEM((RPW, D), jnp.float32),
        pltpu.VMEM_SHARED((NS * RPW, D), jnp.float32),
        [pltpu.SemaphoreType.DMA for _ in range(NBUF)],
        [pltpu.SemaphoreType.DMA for _ in range(NBUF)],
    ],
)


BLK = 512


def _proj_body(x_ref, w_ref, b_ref, o_ref):
    y = jnp.dot(x_ref[...], w_ref[...], preferred_element_type=jnp.float32)
    y = y * (1.0 / L) + b_ref[...]
    s = jnp.sum(y * y, axis=1, keepdims=True)
    o_ref[...] = y * lax.rsqrt(s)


_proj = pl.pallas_call(
    _proj_body,
    grid=(B // BLK,),
    in_specs=[
        pl.BlockSpec((BLK, D), lambda i: (i, 0)),
        pl.BlockSpec((D, D), lambda i: (0, 0)),
        pl.BlockSpec((1, D), lambda i: (0, 0)),
    ],
    out_specs=pl.BlockSpec((BLK, D), lambda i: (i, 0)),
    out_shape=jax.ShapeDtypeStruct((B, D), jnp.float32),
)


_DEST_PATTERN = (
    (np.arange(NCHUNK * IDXC, dtype=np.int32) // L)[None, :]
    + (np.arange(NS, dtype=np.int32) * RPW)[:, None]
).reshape(NS, NCHUNK, IDXC)


def kernel(inputs, table, W, b):
    idx = inputs.astype(jnp.int32).reshape(NW, NCHUNK, IDXC)
    pooled = _pool(idx, jnp.asarray(_DEST_PATTERN), table)
    return _proj(pooled, W, b.reshape(1, D))


# speedup vs baseline: 2.0782x; 1.0093x over previous
import functools

import jax
import jax.numpy as jnp
import numpy as np
from jax import lax
from jax.experimental import pallas as pl
from jax.experimental.pallas import tpu as pltpu
from jax.experimental.pallas import tpu_sc as plsc

D = 128
B = 4096
L = 20

NC, NS = 2, 16
NW = NC * NS
RPW = B // NW
IDXC = 128
NCHUNK = (RPW * L) // IDXC


NBUF = 5
PREF = 3


def _pool_body(idx_hbm, dpat_hbm, table_hbm, out_hbm, idx_v, dest_v, bufs,
               out_v, acc_sh, gsems, ssems):
    sid = lax.axis_index("s")
    wid = sid * NC + lax.axis_index("c")
    pltpu.sync_copy(idx_hbm.at[wid], idx_v)
    for p in range(PREF):
        pltpu.async_copy(table_hbm.at[idx_v.at[p]], bufs[p], gsems[p])
    pltpu.sync_copy(dpat_hbm.at[sid], dest_v)
    zeros = jnp.zeros((16,), jnp.float32)
    for r in range(RPW):
        for d in range(D // 16):
            out_v[r, pl.ds(d * 16, 16)] = zeros
    pltpu.sync_copy(out_v, acc_sh.at[pl.ds(sid * RPW, RPW)])

    def step(j, k):
        b, bn = k % NBUF, (k + PREF) % NBUF
        pltpu.make_async_copy(table_hbm.at[idx_v.at[j]], bufs[b],
                              gsems[b]).wait()
        pltpu.async_copy(bufs[b], acc_sh.at[dest_v.at[j]], ssems[b], add=True)

        @pl.when(j + PREF < NCHUNK)
        def _():
            @pl.when(j + PREF >= NBUF)
            def _():
                pltpu.make_async_copy(
                    bufs[bn], acc_sh.at[dest_v.at[j + PREF - NBUF]],
                    ssems[bn]).wait()
            pltpu.async_copy(table_hbm.at[idx_v.at[j + PREF]], bufs[bn],
                             gsems[bn])

    def ring_body(i, carry):
        for k in range(NBUF):
            step(NBUF * i + k, k)
        return carry

    lax.fori_loop(0, NCHUNK // NBUF, ring_body, 0)
    for jj in range(NCHUNK - NBUF, NCHUNK):
        pltpu.make_async_copy(bufs[jj % NBUF], acc_sh.at[dest_v.at[jj]],
                              ssems[jj % NBUF]).wait()
    pltpu.sync_copy(acc_sh.at[pl.ds(sid * RPW, RPW)],
                    out_hbm.at[pl.ds(wid * RPW, RPW)])


_pool = pl.kernel(
    _pool_body,
    out_type=jax.ShapeDtypeStruct((B, D), jnp.float32),
    mesh=plsc.VectorSubcoreMesh(core_axis_name="c", subcore_axis_name="s"),
    scratch_types=[
        pltpu.VMEM((NCHUNK, IDXC), jnp.int32),
        pltpu.VMEM((NCHUNK, IDXC), jnp.int32),
        [pltpu.VMEM((IDXC, D), jnp.float32) for _ in range(NBUF)],
        pltpu.VMEM((RPW, D), jnp.float32),
        pltpu.VMEM_SHARED((NS * RPW, D), jnp.float32),
        [pltpu.SemaphoreType.DMA for _ in range(NBUF)],
        [pltpu.SemaphoreType.DMA for _ in range(NBUF)],
    ],
)


BLK = 512


def _proj_body(x_ref, w_ref, b_ref, o_ref):
    y = jnp.dot(x_ref[...], w_ref[...], preferred_element_type=jnp.float32)
    y = y * (1.0 / L) + b_ref[...]
    s = jnp.sum(y * y, axis=1, keepdims=True)
    o_ref[...] = y * lax.rsqrt(s)


_proj = pl.pallas_call(
    _proj_body,
    grid=(B // BLK,),
    in_specs=[
        pl.BlockSpec((BLK, D), lambda i: (i, 0)),
        pl.BlockSpec((D, D), lambda i: (0, 0)),
        pl.BlockSpec((1, D), lambda i: (0, 0)),
    ],
    out_specs=pl.BlockSpec((BLK, D), lambda i: (i, 0)),
    out_shape=jax.ShapeDtypeStruct((B, D), jnp.float32),
)


_DEST_PATTERN = (
    (np.arange(NCHUNK * IDXC, dtype=np.int32) // L)[None, :]
    + (np.arange(NS, dtype=np.int32) * RPW)[:, None]
).reshape(NS, NCHUNK, IDXC)


def kernel(inputs, table, W, b):
    idx = inputs.astype(jnp.int32).reshape(NW, NCHUNK, IDXC)
    pooled = _pool(idx, jnp.asarray(_DEST_PATTERN), table)
    return _proj(pooled, W, b.reshape(1, D))


# speedup vs baseline: 2.1554x; 1.0372x over previous
import functools

import jax
import jax.numpy as jnp
import numpy as np
from jax import lax
from jax.experimental import pallas as pl
from jax.experimental.pallas import tpu as pltpu
from jax.experimental.pallas import tpu_sc as plsc

D = 128
B = 4096
L = 20

NC, NS = 2, 16
NW = NC * NS
RPW = B // NW
IDXC = 128
NCHUNK = (RPW * L) // IDXC


NBUF = 5
PREF = 3


def _pool_body(idx_hbm, dpat_hbm, table_hbm, out_hbm, idx_v, dest_v, bufs,
               out_v, acc_sh, gsems, ssems):
    sid = lax.axis_index("s")
    wid = sid * NC + lax.axis_index("c")
    pltpu.sync_copy(idx_hbm.at[wid], idx_v)
    for p in range(PREF):
        pltpu.async_copy(table_hbm.at[idx_v.at[p]], bufs[p], gsems[p])
    pltpu.sync_copy(dpat_hbm.at[sid], dest_v)
    zeros = jnp.zeros((16,), jnp.float32)
    for r in range(RPW):
        for d in range(D // 16):
            out_v[r, pl.ds(d * 16, 16)] = zeros
    pltpu.sync_copy(out_v, acc_sh.at[pl.ds(sid * RPW, RPW)])

    def step(j, k):
        b, bn = k % NBUF, (k + PREF) % NBUF
        pltpu.make_async_copy(table_hbm.at[idx_v.at[j]], bufs[b],
                              gsems[b]).wait()
        pltpu.async_copy(bufs[b], acc_sh.at[dest_v.at[j]], ssems[b], add=True)

        @pl.when(j + PREF < NCHUNK)
        def _():
            @pl.when(j + PREF >= NBUF)
            def _():
                pltpu.make_async_copy(
                    bufs[bn], acc_sh.at[dest_v.at[j + PREF - NBUF]],
                    ssems[bn]).wait()
            pltpu.async_copy(table_hbm.at[idx_v.at[j + PREF]], bufs[bn],
                             gsems[bn])

    def ring_body(i, carry):
        for k in range(NBUF):
            step(NBUF * i + k, k)
        return carry

    lax.fori_loop(0, NCHUNK // NBUF, ring_body, 0)
    for jj in range(NCHUNK - NBUF, NCHUNK):
        pltpu.make_async_copy(bufs[jj % NBUF], acc_sh.at[dest_v.at[jj]],
                              ssems[jj % NBUF]).wait()
    pltpu.sync_copy(acc_sh.at[pl.ds(sid * RPW, RPW)],
                    out_hbm.at[pl.ds(wid * RPW, RPW)])


_pool = pl.kernel(
    _pool_body,
    out_type=jax.ShapeDtypeStruct((B, D), jnp.float32),
    mesh=plsc.VectorSubcoreMesh(core_axis_name="c", subcore_axis_name="s"),
    scratch_types=[
        pltpu.VMEM((NCHUNK, IDXC), jnp.int32),
        pltpu.VMEM((NCHUNK, IDXC), jnp.int32),
        [pltpu.VMEM((IDXC, D), jnp.float32) for _ in range(NBUF)],
        pltpu.VMEM((RPW, D), jnp.float32),
        pltpu.VMEM_SHARED((NS * RPW, D), jnp.float32),
        [pltpu.SemaphoreType.DMA for _ in range(NBUF)],
        [pltpu.SemaphoreType.DMA for _ in range(NBUF)],
    ],
)


BLK = 1024


def _proj_body(x_ref, w_ref, b_ref, o_ref):
    y = jnp.dot(x_ref[...], w_ref[...], preferred_element_type=jnp.float32)
    y = y * (1.0 / L) + b_ref[...]
    s = jnp.sum(y * y, axis=1, keepdims=True)
    o_ref[...] = y * lax.rsqrt(s)


_proj = pl.pallas_call(
    _proj_body,
    grid=(B // BLK,),
    in_specs=[
        pl.BlockSpec((BLK, D), lambda i: (i, 0)),
        pl.BlockSpec((D, D), lambda i: (0, 0)),
        pl.BlockSpec((1, D), lambda i: (0, 0)),
    ],
    out_specs=pl.BlockSpec((BLK, D), lambda i: (i, 0)),
    out_shape=jax.ShapeDtypeStruct((B, D), jnp.float32),
)


_DEST_PATTERN = (
    (np.arange(NCHUNK * IDXC, dtype=np.int32) // L)[None, :]
    + (np.arange(NS, dtype=np.int32) * RPW)[:, None]
).reshape(NS, NCHUNK, IDXC)


def kernel(inputs, table, W, b):
    idx = inputs.astype(jnp.int32).reshape(NW, NCHUNK, IDXC)
    pooled = _pool(idx, jnp.asarray(_DEST_PATTERN), table)
    return _proj(pooled, W, b.reshape(1, D))


# speedup vs baseline: 2.1610x; 1.0026x over previous
import jax
import jax.numpy as jnp
import numpy as np
from jax import lax
from jax.experimental import pallas as pl
from jax.experimental.pallas import tpu as pltpu
from jax.experimental.pallas import tpu_sc as plsc

D = 128
B = 4096
L = 20

NC, NS = 2, 16
NW = NC * NS
RPW = B // NW
IDXC = 128
NCHUNK = (RPW * L) // IDXC


NBUF = 5
PREF = 3


def _pool_body(idx_hbm, dpat_hbm, table_hbm, out_hbm, idx_v, dest_v, bufs,
               out_v, acc_sh, gsems, ssems):
    sid = lax.axis_index("s")
    wid = sid * NC + lax.axis_index("c")
    pltpu.sync_copy(idx_hbm.at[wid], idx_v)
    for p in range(PREF):
        pltpu.async_copy(table_hbm.at[idx_v.at[p]], bufs[p], gsems[p])
    pltpu.sync_copy(dpat_hbm.at[sid], dest_v)
    zeros = jnp.zeros((16,), jnp.float32)
    for r in range(RPW):
        for d in range(D // 16):
            out_v[r, pl.ds(d * 16, 16)] = zeros
    pltpu.sync_copy(out_v, acc_sh.at[pl.ds(sid * RPW, RPW)])

    def step(j, k):
        b, bn = k % NBUF, (k + PREF) % NBUF
        pltpu.make_async_copy(table_hbm.at[idx_v.at[j]], bufs[b],
                              gsems[b]).wait()
        pltpu.async_copy(bufs[b], acc_sh.at[dest_v.at[j]], ssems[b], add=True)

        @pl.when(j + PREF < NCHUNK)
        def _():
            @pl.when(j + PREF >= NBUF)
            def _():
                pltpu.make_async_copy(
                    bufs[bn], acc_sh.at[dest_v.at[j + PREF - NBUF]],
                    ssems[bn]).wait()
            pltpu.async_copy(table_hbm.at[idx_v.at[j + PREF]], bufs[bn],
                             gsems[bn])

    def ring_body(i, carry):
        for k in range(NBUF):
            step(NBUF * i + k, k)
        return carry

    lax.fori_loop(0, NCHUNK // NBUF, ring_body, 0)
    for jj in range(NCHUNK - NBUF, NCHUNK):
        pltpu.make_async_copy(bufs[jj % NBUF], acc_sh.at[dest_v.at[jj]],
                              ssems[jj % NBUF]).wait()
    pltpu.sync_copy(acc_sh.at[pl.ds(sid * RPW, RPW)],
                    out_hbm.at[pl.ds(wid * RPW, RPW)])


_pool = pl.kernel(
    _pool_body,
    out_type=jax.ShapeDtypeStruct((B, D), jnp.float32),
    mesh=plsc.VectorSubcoreMesh(core_axis_name="c", subcore_axis_name="s"),
    scratch_types=[
        pltpu.VMEM((NCHUNK, IDXC), jnp.int32),
        pltpu.VMEM((NCHUNK, IDXC), jnp.int32),
        [pltpu.VMEM((IDXC, D), jnp.float32) for _ in range(NBUF)],
        pltpu.VMEM((RPW, D), jnp.float32),
        pltpu.VMEM_SHARED((NS * RPW, D), jnp.float32),
        [pltpu.SemaphoreType.DMA for _ in range(NBUF)],
        [pltpu.SemaphoreType.DMA for _ in range(NBUF)],
    ],
)


BLK = 1024


def _proj_body(x_ref, w_ref, b_ref, o_ref):
    y = jnp.dot(x_ref[...], w_ref[...], preferred_element_type=jnp.float32)
    y = y * (1.0 / L) + b_ref[...]
    s = jnp.sum(y * y, axis=1, keepdims=True)
    o_ref[...] = y * lax.rsqrt(s)


_proj = pl.pallas_call(
    _proj_body,
    grid=(B // BLK,),
    in_specs=[
        pl.BlockSpec((BLK, D), lambda i: (i, 0)),
        pl.BlockSpec((D, D), lambda i: (0, 0)),
        pl.BlockSpec((1, D), lambda i: (0, 0)),
    ],
    out_specs=pl.BlockSpec((BLK, D), lambda i: (i, 0)),
    out_shape=jax.ShapeDtypeStruct((B, D), jnp.float32),
)


_DEST_PATTERN = (
    (np.arange(NCHUNK * IDXC, dtype=np.int32) // L)[None, :]
    + (np.arange(NS, dtype=np.int32) * RPW)[:, None]
).reshape(NS, NCHUNK, IDXC)


def kernel(inputs, table, W, b):
    idx = inputs.astype(jnp.int32).reshape(NW, NCHUNK, IDXC)
    pooled = _pool(idx, jnp.asarray(_DEST_PATTERN), table)
    return _proj(pooled, W, b.reshape(1, D))
